# baseline (speedup 1.0000x reference)
import functools
import jax
import jax.numpy as jnp
from jax.experimental import pallas as pl
from jax.experimental.pallas import tpu as pltpu

M = 8192
N = 32
Lp = 5
D = 768
PD = 768
B = 64

CHUNK = 1024
NCH = M // CHUNK
NEG = -3.0e38


def _select_body(x_ref, keys_ref, sel_ref, loss_ref, xn_ref, sim_ref, work_ref,
                 keyw_ref):
    ci = pl.program_id(0)

    @pl.when(ci == 0)
    def _():
        xx = x_ref[...]
        nrm = jnp.sqrt(jnp.sum(xx * xx, axis=1, keepdims=True))
        xn_ref[...] = xx / jnp.maximum(nrm, 1e-12)

    kk = keys_ref[...]
    knrm = jnp.sqrt(jnp.sum(kk * kk, axis=1, keepdims=True))
    kn = kk / jnp.maximum(knrm, 1e-12)
    sim_ref[:, pl.ds(ci * CHUNK, CHUNK)] = jnp.dot(
        xn_ref[...], kn.T, preferred_element_type=jnp.float32)

    @pl.when(ci == NCH - 1)
    def _():
        sim = sim_ref[...]
        work_ref[...] = sim

        def body_a(_, thr):
            m = jnp.max(work_ref[...], axis=1, keepdims=True)
            work_ref[...] = jnp.where(work_ref[...] >= m, NEG, work_ref[...])
            return m

        thr = jax.lax.fori_loop(0, N, body_a,
                                jnp.zeros((B, 1), jnp.float32))

        topmask = sim >= thr
        counts = jnp.sum(topmask.astype(jnp.int32), axis=0, keepdims=True)
        colsum = jnp.sum(sim, axis=0, keepdims=True)
        ids = jax.lax.broadcasted_iota(jnp.int32, (1, M), 1)
        keyw_ref[...] = counts * 16384 + (M - 1 - ids)

        def body_b(n, acc):
            kw = keyw_ref[...]
            mk = jnp.max(kw)
            hit = kw == mk
            acc = acc + jnp.sum(jnp.where(hit, colsum, 0.0))
            keyw_ref[...] = jnp.where(hit, -1, kw)
            sel_ref[0, n] = (M - 1) - (mk & 16383)
            return acc

        acc = jax.lax.fori_loop(0, N, body_b, jnp.float32(0.0))
        loss_ref[0, 0] = acc / B


BBLK = 8


def _gather_body(sel_sref, *refs):
    del sel_sref
    pool_refs = refs[:N]
    out_ref = refs[N]
    for i in range(N):
        out_ref[:, i * Lp:(i + 1) * Lp, :] = jnp.broadcast_to(
            pool_refs[i][0], (BBLK, Lp, D))


@jax.jit
def kernel(x, prompt_pool, prompt_keys):
    sel, loss = pl.pallas_call(
        _select_body,
        grid=(NCH,),
        in_specs=[
            pl.BlockSpec((B, PD), lambda c: (0, 0)),
            pl.BlockSpec((CHUNK, PD), lambda c: (c, 0)),
        ],
        out_specs=[
            pl.BlockSpec(memory_space=pltpu.SMEM),
            pl.BlockSpec(memory_space=pltpu.SMEM),
        ],
        out_shape=[
            jax.ShapeDtypeStruct((1, N), jnp.int32),
            jax.ShapeDtypeStruct((1, 1), jnp.float32),
        ],
        scratch_shapes=[
            pltpu.VMEM((B, PD), jnp.float32),
            pltpu.VMEM((B, M), jnp.float32),
            pltpu.VMEM((B, M), jnp.float32),
            pltpu.VMEM((1, M), jnp.int32),
        ],
    )(x, prompt_keys)

    sel_flat = sel.reshape((N,))

    out = pl.pallas_call(
        _gather_body,
        grid_spec=pltpu.PrefetchScalarGridSpec(
            num_scalar_prefetch=1,
            grid=(B // BBLK,),
            in_specs=[
                pl.BlockSpec((1, Lp, D),
                             lambda g, sref, i=i: (sref[i], 0, 0))
                for i in range(N)
            ],
            out_specs=pl.BlockSpec((BBLK, N * Lp, D),
                                   lambda g, sref: (g, 0, 0)),
        ),
        out_shape=jax.ShapeDtypeStruct((B, N * Lp, D), jnp.float32),
    )(sel_flat, *([prompt_pool] * N))

    return out, loss.reshape(())
